# baseline (speedup 1.0000x reference)
import jax
import jax.numpy as jnp
import numpy as np
from jax.experimental import pallas as pl

B, S, H = 128, 8192, 128
NSUB = 2
CS = S // NSUB
NEG = float(np.log(np.float32(1e-45)))


def _dotg(a, b, dims):
    return jax.lax.dot_general(a.astype(jnp.bfloat16), b.astype(jnp.bfloat16),
                               (dims, ((), ())),
                               preferred_element_type=jnp.float32)


def _attn_kernel(mask_ref, userc_ref, x4_ref, wenc_ref, benc_ref,
                 w1_ref, w2_ref, vt_ref, out_p_ref, out_i_ref):
    f32 = jnp.float32
    b = pl.program_id(0)

    wenc = wenc_ref[...]
    bencT = benc_ref[...]
    w1 = w1_ref[...]
    vt8 = vt_ref[...]
    dtT = _dotg(w2_ref[...], userc_ref[0],
                ((1,), (0,)))[:, 0:1]

    encs = []
    for j in range(NSUB):
        x4_j = x4_ref[0][:, j * CS:(j + 1) * CS]
        encs.append(_dotg(wenc, x4_j, ((1,), (0,))) + bencT)
    ts = [jnp.tanh(_dotg(w1, e, ((1,), (0,))) + dtT) for e in encs]
    us = [_dotg(vt8, t, ((1,), (0,)))[0:1, :] for t in ts]

    score = jnp.where(mask_ref[0] != 0,
                      jnp.concatenate(us, axis=0), f32(NEG)) * f32(10.0)

    m = jnp.max(score)
    se = jnp.sum(jnp.exp(score - m))
    iota = (jax.lax.broadcasted_iota(jnp.int32, (NSUB, CS), 0) * CS
            + jax.lax.broadcasted_iota(jnp.int32, (NSUB, CS), 1))
    ix = jnp.min(jnp.where(score == m, iota, jnp.int32(S)))

    rowsel = jax.lax.broadcasted_iota(jnp.int32, (8, 1), 0) == (b % 8)
    out_p_ref[...] = jnp.where(rowsel, f32(1.0) / se, out_p_ref[...])
    out_i_ref[...] = jnp.where(rowsel, ix, out_i_ref[...])


@jax.jit
def kernel(mask, user, static_server_seq, tmp_server_capacity, server_active,
           Wenc, benc, W1, W2, vt):
    x4 = jnp.stack([static_server_seq[:, :, 0], static_server_seq[:, :, 1],
                    tmp_server_capacity[:, :, 0], server_active[:, :, 0]],
                   axis=1)
    mask3 = mask.reshape(B, NSUB, CS).astype(jnp.float32)
    userc8 = jnp.broadcast_to(user.reshape(B, H, 1), (B, H, 8))
    bencT = benc.reshape(H, 1)
    vt8 = jnp.broadcast_to(vt, (8, H))

    row3 = lambda b: (b, 0, 0)
    rep2 = lambda b: (0, 0)

    top_p, top_i = pl.pallas_call(
        _attn_kernel,
        grid=(B,),
        in_specs=[
            pl.BlockSpec((1, NSUB, CS), row3),
            pl.BlockSpec((1, H, 8), row3),
            pl.BlockSpec((1, 4, S), row3),
            pl.BlockSpec((H, 4), rep2),
            pl.BlockSpec((H, 1), rep2),
            pl.BlockSpec((H, H), rep2),
            pl.BlockSpec((H, H), rep2),
            pl.BlockSpec((8, H), rep2),
        ],
        out_specs=[
            pl.BlockSpec((8, 1), lambda b: (b // 8, 0)),
            pl.BlockSpec((8, 1), lambda b: (b // 8, 0)),
        ],
        out_shape=[
            jax.ShapeDtypeStruct((B, 1), jnp.float32),
            jax.ShapeDtypeStruct((B, 1), jnp.int32),
        ],
    )(mask3, userc8, x4, Wenc, bencT, W1, W2, vt8)

    return (top_p.reshape(B), top_i.reshape(B))


# speedup vs baseline: 1.1048x; 1.1048x over previous
import jax
import jax.numpy as jnp
import numpy as np
from jax.experimental import pallas as pl

B, S, H = 128, 8192, 128
BT = 2
NSUB = 4
CS = S // NSUB
NEG = float(np.log(np.float32(1e-45)))


def _dotg(a, b, dims):
    return jax.lax.dot_general(a.astype(jnp.bfloat16), b.astype(jnp.bfloat16),
                               (dims, ((), ())),
                               preferred_element_type=jnp.float32)


def _attn_kernel(mask_ref, userc_ref, x4_ref, wenc_ref, benc_ref,
                 w1_ref, w2_ref, vt_ref, out_p_ref, out_i_ref):
    f32 = jnp.float32
    b = pl.program_id(0)

    wenc = wenc_ref[...]
    bencT = benc_ref[...]
    w1 = w1_ref[...]
    vt8 = vt_ref[...]

    encs = []
    for i in range(BT):
        for j in range(NSUB):
            x4_j = x4_ref[i][:, j * CS:(j + 1) * CS]
            encs.append(_dotg(wenc, x4_j, ((1,), (0,))) + bencT)
    dts = [_dotg(w2_ref[...], userc_ref[i], ((1,), (0,)))[:, 0:1]
           for i in range(BT)]
    ts = [jnp.tanh(_dotg(w1, e, ((1,), (0,))) + dts[k // NSUB])
          for k, e in enumerate(encs)]
    us = [_dotg(vt8, t, ((1,), (0,)))[0:1, :] for t in ts]

    iota = (jax.lax.broadcasted_iota(jnp.int32, (NSUB, CS), 0) * CS
            + jax.lax.broadcasted_iota(jnp.int32, (NSUB, CS), 1))
    row_iota = jax.lax.broadcasted_iota(jnp.int32, (8, 1), 0)
    p_upd, i_upd = out_p_ref[...], out_i_ref[...]
    for i in range(BT):
        score = jnp.where(mask_ref[i] != 0,
                          jnp.concatenate(us[i * NSUB:(i + 1) * NSUB], axis=0),
                          f32(NEG)) * f32(10.0)
        m = jnp.max(score)
        se = jnp.sum(jnp.exp(score - m))
        ix = jnp.min(jnp.where(score == m, iota, jnp.int32(S)))
        rowsel = row_iota == (BT * (b % (8 // BT)) + i)
        p_upd = jnp.where(rowsel, f32(1.0) / se, p_upd)
        i_upd = jnp.where(rowsel, ix, i_upd)
    out_p_ref[...] = p_upd
    out_i_ref[...] = i_upd


@jax.jit
def kernel(mask, user, static_server_seq, tmp_server_capacity, server_active,
           Wenc, benc, W1, W2, vt):
    x4 = jnp.stack([static_server_seq[:, :, 0], static_server_seq[:, :, 1],
                    tmp_server_capacity[:, :, 0], server_active[:, :, 0]],
                   axis=1)
    mask3 = mask.reshape(B, NSUB, CS).astype(jnp.float32)
    userc8 = jnp.broadcast_to(user.reshape(B, H, 1), (B, H, 8))
    bencT = benc.reshape(H, 1)
    vt8 = jnp.broadcast_to(vt, (8, H))

    row3 = lambda b: (b, 0, 0)
    rep2 = lambda b: (0, 0)

    top_p, top_i = pl.pallas_call(
        _attn_kernel,
        grid=(B // BT,),
        in_specs=[
            pl.BlockSpec((BT, NSUB, CS), row3),
            pl.BlockSpec((BT, H, 8), row3),
            pl.BlockSpec((BT, 4, S), row3),
            pl.BlockSpec((H, 4), rep2),
            pl.BlockSpec((H, 1), rep2),
            pl.BlockSpec((H, H), rep2),
            pl.BlockSpec((H, H), rep2),
            pl.BlockSpec((8, H), rep2),
        ],
        out_specs=[
            pl.BlockSpec((8, 1), lambda b: (b // (8 // BT), 0)),
            pl.BlockSpec((8, 1), lambda b: (b // (8 // BT), 0)),
        ],
        out_shape=[
            jax.ShapeDtypeStruct((B, 1), jnp.float32),
            jax.ShapeDtypeStruct((B, 1), jnp.int32),
        ],
    )(mask3, userc8, x4, Wenc, bencT, W1, W2, vt8)

    return (top_p.reshape(B), top_i.reshape(B))


# speedup vs baseline: 1.1835x; 1.0713x over previous
import jax
import jax.numpy as jnp
import numpy as np
from jax.experimental import pallas as pl

B, S, H = 128, 8192, 128
BT = 4
NSUB = 4
CS = S // NSUB
NEG = float(np.log(np.float32(1e-45)))


def _dotg(a, b, dims):
    return jax.lax.dot_general(a.astype(jnp.bfloat16), b.astype(jnp.bfloat16),
                               (dims, ((), ())),
                               preferred_element_type=jnp.float32)


def _attn_kernel(mask_ref, userc_ref, x4_ref, wenc_ref, benc_ref,
                 w1_ref, w2_ref, vt_ref, out_p_ref, out_i_ref):
    f32 = jnp.float32
    b = pl.program_id(0)

    wenc = wenc_ref[...]
    bencT = benc_ref[...]
    w1 = w1_ref[...]
    vt8 = vt_ref[...]

    encs = []
    for i in range(BT):
        for j in range(NSUB):
            x4_j = x4_ref[i][:, j * CS:(j + 1) * CS]
            encs.append(_dotg(wenc, x4_j, ((1,), (0,))) + bencT)
    dts = [_dotg(w2_ref[...], userc_ref[i], ((1,), (0,)))[:, 0:1]
           for i in range(BT)]
    ts = [jnp.tanh(_dotg(w1, e, ((1,), (0,))) + dts[k // NSUB])
          for k, e in enumerate(encs)]
    us = [_dotg(vt8, t, ((1,), (0,)))[0:1, :] for t in ts]

    iota = (jax.lax.broadcasted_iota(jnp.int32, (NSUB, CS), 0) * CS
            + jax.lax.broadcasted_iota(jnp.int32, (NSUB, CS), 1))
    row_iota = jax.lax.broadcasted_iota(jnp.int32, (8, 1), 0)
    p_upd, i_upd = out_p_ref[...], out_i_ref[...]
    for i in range(BT):
        score = jnp.where(mask_ref[i] != 0,
                          jnp.concatenate(us[i * NSUB:(i + 1) * NSUB], axis=0),
                          f32(NEG)) * f32(10.0)
        m = jnp.max(score)
        se = jnp.sum(jnp.exp(score - m))
        ix = jnp.min(jnp.where(score == m, iota, jnp.int32(S)))
        rowsel = row_iota == (BT * (b % (8 // BT)) + i)
        p_upd = jnp.where(rowsel, f32(1.0) / se, p_upd)
        i_upd = jnp.where(rowsel, ix, i_upd)
    out_p_ref[...] = p_upd
    out_i_ref[...] = i_upd


@jax.jit
def kernel(mask, user, static_server_seq, tmp_server_capacity, server_active,
           Wenc, benc, W1, W2, vt):
    x4 = jnp.stack([static_server_seq[:, :, 0], static_server_seq[:, :, 1],
                    tmp_server_capacity[:, :, 0], server_active[:, :, 0]],
                   axis=1)
    mask3 = mask.reshape(B, NSUB, CS).astype(jnp.float32)
    userc8 = jnp.broadcast_to(user.reshape(B, H, 1), (B, H, 8))
    bencT = benc.reshape(H, 1)
    vt8 = jnp.broadcast_to(vt, (8, H))

    row3 = lambda b: (b, 0, 0)
    rep2 = lambda b: (0, 0)

    top_p, top_i = pl.pallas_call(
        _attn_kernel,
        grid=(B // BT,),
        in_specs=[
            pl.BlockSpec((BT, NSUB, CS), row3),
            pl.BlockSpec((BT, H, 8), row3),
            pl.BlockSpec((BT, 4, S), row3),
            pl.BlockSpec((H, 4), rep2),
            pl.BlockSpec((H, 1), rep2),
            pl.BlockSpec((H, H), rep2),
            pl.BlockSpec((H, H), rep2),
            pl.BlockSpec((8, H), rep2),
        ],
        out_specs=[
            pl.BlockSpec((8, 1), lambda b: (b // (8 // BT), 0)),
            pl.BlockSpec((8, 1), lambda b: (b // (8 // BT), 0)),
        ],
        out_shape=[
            jax.ShapeDtypeStruct((B, 1), jnp.float32),
            jax.ShapeDtypeStruct((B, 1), jnp.int32),
        ],
    )(mask3, userc8, x4, Wenc, bencT, W1, W2, vt8)

    return (top_p.reshape(B), top_i.reshape(B))


# speedup vs baseline: 1.1942x; 1.0090x over previous
import jax
import jax.numpy as jnp
import numpy as np
from jax.experimental import pallas as pl

B, S, H = 128, 8192, 128
BT = 8
NSUB = 4
CS = S // NSUB
NEG = float(np.log(np.float32(1e-45)))


def _dotg(a, b, dims):
    return jax.lax.dot_general(a.astype(jnp.bfloat16), b.astype(jnp.bfloat16),
                               (dims, ((), ())),
                               preferred_element_type=jnp.float32)


def _attn_kernel(mask_ref, userc_ref, x4_ref, wenc_ref, benc_ref,
                 w1_ref, w2_ref, vt_ref, out_p_ref, out_i_ref):
    f32 = jnp.float32
    b = pl.program_id(0)

    wenc = wenc_ref[...]
    bencT = benc_ref[...]
    w1 = w1_ref[...]
    vt8 = vt_ref[...]

    encs = []
    for i in range(BT):
        for j in range(NSUB):
            x4_j = x4_ref[i][:, j * CS:(j + 1) * CS]
            encs.append(_dotg(wenc, x4_j, ((1,), (0,))) + bencT)
    dts = [_dotg(w2_ref[...], userc_ref[i], ((1,), (0,)))[:, 0:1]
           for i in range(BT)]
    ts = [jnp.tanh(_dotg(w1, e, ((1,), (0,))) + dts[k // NSUB])
          for k, e in enumerate(encs)]
    us = [_dotg(vt8, t, ((1,), (0,)))[0:1, :] for t in ts]

    iota = (jax.lax.broadcasted_iota(jnp.int32, (NSUB, CS), 0) * CS
            + jax.lax.broadcasted_iota(jnp.int32, (NSUB, CS), 1))
    row_iota = jax.lax.broadcasted_iota(jnp.int32, (8, 1), 0)
    p_upd, i_upd = out_p_ref[...], out_i_ref[...]
    for i in range(BT):
        score = jnp.where(mask_ref[i] != 0,
                          jnp.concatenate(us[i * NSUB:(i + 1) * NSUB], axis=0),
                          f32(NEG)) * f32(10.0)
        m = jnp.max(score)
        se = jnp.sum(jnp.exp(score - m))
        ix = jnp.min(jnp.where(score == m, iota, jnp.int32(S)))
        rowsel = row_iota == (BT * (b % (8 // BT)) + i)
        p_upd = jnp.where(rowsel, f32(1.0) / se, p_upd)
        i_upd = jnp.where(rowsel, ix, i_upd)
    out_p_ref[...] = p_upd
    out_i_ref[...] = i_upd


@jax.jit
def kernel(mask, user, static_server_seq, tmp_server_capacity, server_active,
           Wenc, benc, W1, W2, vt):
    x4 = jnp.stack([static_server_seq[:, :, 0], static_server_seq[:, :, 1],
                    tmp_server_capacity[:, :, 0], server_active[:, :, 0]],
                   axis=1)
    mask3 = mask.reshape(B, NSUB, CS).astype(jnp.float32)
    userc8 = jnp.broadcast_to(user.reshape(B, H, 1), (B, H, 8))
    bencT = benc.reshape(H, 1)
    vt8 = jnp.broadcast_to(vt, (8, H))

    row3 = lambda b: (b, 0, 0)
    rep2 = lambda b: (0, 0)

    top_p, top_i = pl.pallas_call(
        _attn_kernel,
        grid=(B // BT,),
        in_specs=[
            pl.BlockSpec((BT, NSUB, CS), row3),
            pl.BlockSpec((BT, H, 8), row3),
            pl.BlockSpec((BT, 4, S), row3),
            pl.BlockSpec((H, 4), rep2),
            pl.BlockSpec((H, 1), rep2),
            pl.BlockSpec((H, H), rep2),
            pl.BlockSpec((H, H), rep2),
            pl.BlockSpec((8, H), rep2),
        ],
        out_specs=[
            pl.BlockSpec((8, 1), lambda b: (b // (8 // BT), 0)),
            pl.BlockSpec((8, 1), lambda b: (b // (8 // BT), 0)),
        ],
        out_shape=[
            jax.ShapeDtypeStruct((B, 1), jnp.float32),
            jax.ShapeDtypeStruct((B, 1), jnp.int32),
        ],
    )(mask3, userc8, x4, Wenc, bencT, W1, W2, vt8)

    return (top_p.reshape(B), top_i.reshape(B))


# speedup vs baseline: 1.2667x; 1.0607x over previous
import jax
import jax.numpy as jnp
import numpy as np
from jax.experimental import pallas as pl

B, S, H = 128, 8192, 128
BT = 8
NSUB = 4
CS = S // NSUB
NEG = float(np.log(np.float32(1e-45)))


def _dotg(a, b, dims):
    return jax.lax.dot_general(a.astype(jnp.bfloat16), b.astype(jnp.bfloat16),
                               (dims, ((), ())),
                               preferred_element_type=jnp.float32)


def _attn_kernel(mask_ref, userc_ref, x2_ref, cap_ref, act_ref,
                 wenc_ref, benc_ref,
                 w1_ref, w2_ref, vt_ref, out_p_ref, out_i_ref):
    f32 = jnp.float32
    b = pl.program_id(0)

    wenc = wenc_ref[...]
    bencT = benc_ref[...]
    w1 = w1_ref[...]
    vt8 = vt_ref[...]

    encs = []
    for i in range(BT):
        for j in range(NSUB):
            sl = slice(j * CS, (j + 1) * CS)
            x4_j = jnp.concatenate(
                [x2_ref[i][:, sl], cap_ref[i][:, sl], act_ref[i][:, sl]],
                axis=0)
            encs.append(_dotg(wenc, x4_j, ((1,), (0,))) + bencT)
    dts = [_dotg(w2_ref[...], userc_ref[i], ((1,), (0,)))[:, 0:1]
           for i in range(BT)]
    ts = [jnp.tanh(_dotg(w1, e, ((1,), (0,))) + dts[k // NSUB])
          for k, e in enumerate(encs)]
    us = [_dotg(vt8, t, ((1,), (0,)))[0:1, :] for t in ts]

    iota = (jax.lax.broadcasted_iota(jnp.int32, (NSUB, CS), 0) * CS
            + jax.lax.broadcasted_iota(jnp.int32, (NSUB, CS), 1))
    row_iota = jax.lax.broadcasted_iota(jnp.int32, (8, 1), 0)
    p_upd, i_upd = out_p_ref[...], out_i_ref[...]
    for i in range(BT):
        score = jnp.where(mask_ref[i] != 0,
                          jnp.concatenate(us[i * NSUB:(i + 1) * NSUB], axis=0),
                          f32(NEG)) * f32(10.0)
        m = jnp.max(score)
        se = jnp.sum(jnp.exp(score - m))
        ix = jnp.min(jnp.where(score == m, iota, jnp.int32(S)))
        rowsel = row_iota == (BT * (b % (8 // BT)) + i)
        p_upd = jnp.where(rowsel, f32(1.0) / se, p_upd)
        i_upd = jnp.where(rowsel, ix, i_upd)
    out_p_ref[...] = p_upd
    out_i_ref[...] = i_upd


@jax.jit
def kernel(mask, user, static_server_seq, tmp_server_capacity, server_active,
           Wenc, benc, W1, W2, vt):
    x2 = jnp.stack([static_server_seq[:, :, 0], static_server_seq[:, :, 1]],
                   axis=1)
    capR = tmp_server_capacity.reshape(B, 1, S)
    actR = server_active.reshape(B, 1, S)
    mask3 = mask.reshape(B, NSUB, CS).astype(jnp.float32)
    userc8 = jnp.broadcast_to(user.reshape(B, H, 1), (B, H, 8))
    bencT = benc.reshape(H, 1)
    vt8 = jnp.broadcast_to(vt, (8, H))

    row3 = lambda b: (b, 0, 0)
    rep2 = lambda b: (0, 0)

    top_p, top_i = pl.pallas_call(
        _attn_kernel,
        grid=(B // BT,),
        in_specs=[
            pl.BlockSpec((BT, NSUB, CS), row3),
            pl.BlockSpec((BT, H, 8), row3),
            pl.BlockSpec((BT, 2, S), row3),
            pl.BlockSpec((BT, 1, S), row3),
            pl.BlockSpec((BT, 1, S), row3),
            pl.BlockSpec((H, 4), rep2),
            pl.BlockSpec((H, 1), rep2),
            pl.BlockSpec((H, H), rep2),
            pl.BlockSpec((H, H), rep2),
            pl.BlockSpec((8, H), rep2),
        ],
        out_specs=[
            pl.BlockSpec((8, 1), lambda b: (b // (8 // BT), 0)),
            pl.BlockSpec((8, 1), lambda b: (b // (8 // BT), 0)),
        ],
        out_shape=[
            jax.ShapeDtypeStruct((B, 1), jnp.float32),
            jax.ShapeDtypeStruct((B, 1), jnp.int32),
        ],
    )(mask3, userc8, x2, capR, actR, Wenc, bencT, W1, W2, vt8)

    return (top_p.reshape(B), top_i.reshape(B))
